# baseline (speedup 1.0000x reference)
import jax
import jax.numpy as jnp
from jax.experimental import pallas as pl

B, D, H, W_SP = 32, 64, 32, 32
K = 1024
N = H * W_SP


def _vq_kernel(x_ref, emb_ref, q_ref, inds_ref, loss_ref):
    x = x_ref[0].reshape(D, N)
    emb = emb_ref[...]

    scores = jnp.dot(emb, x, preferred_element_type=jnp.float32)
    e_sq = jnp.sum(emb * emb, axis=1, keepdims=True)
    x_sq = jnp.sum(x * x, axis=0, keepdims=True)
    dist = x_sq + e_sq - 2.0 * scores

    min_v = jnp.min(dist, axis=0, keepdims=True)
    iota_k = jax.lax.broadcasted_iota(jnp.int32, (K, N), 0)
    inds = jnp.min(jnp.where(dist == min_v, iota_k, K), axis=0,
                   keepdims=True)

    one_hot = (iota_k == inds).astype(jnp.float32)
    quantized = jax.lax.dot_general(
        emb, one_hot, (((0,), (0,)), ((), ())),
        preferred_element_type=jnp.float32)

    q_ref[0] = quantized.reshape(D, H, W_SP)
    inds_ref[...] = inds.reshape(1, 1, N)
    diff = quantized - x
    loss_ref[...] = jnp.sum(diff * diff).reshape(1, 1, 1)


def kernel(latents, emb):
    q_out, inds, loss_parts = pl.pallas_call(
        _vq_kernel,
        grid=(B,),
        in_specs=[
            pl.BlockSpec((1, D, H, W_SP), lambda b: (b, 0, 0, 0)),
            pl.BlockSpec((K, D), lambda b: (0, 0)),
        ],
        out_specs=[
            pl.BlockSpec((1, D, H, W_SP), lambda b: (b, 0, 0, 0)),
            pl.BlockSpec((1, 1, N), lambda b: (b, 0, 0)),
            pl.BlockSpec((1, 1, 1), lambda b: (b, 0, 0)),
        ],
        out_shape=[
            jax.ShapeDtypeStruct((B, D, H, W_SP), jnp.float32),
            jax.ShapeDtypeStruct((B, 1, N), jnp.int32),
            jax.ShapeDtypeStruct((B, 1, 1), jnp.float32),
        ],
    )(latents, emb)
    loss = jnp.sum(loss_parts) / (B * N * D)
    encoding_inds = inds.reshape(B * N)
    return (q_out, loss, loss, encoding_inds)


# speedup vs baseline: 1.4238x; 1.4238x over previous
import jax
import jax.numpy as jnp
from jax.experimental import pallas as pl
from jax.experimental.pallas import tpu as pltpu

B, D, H, W_SP = 32, 64, 32, 32
K = 1024
N = H * W_SP


def _vq_kernel(x_ref, emb_ref, q_ref, inds_ref, loss_ref):
    x = x_ref[0]
    emb = emb_ref[...]

    scores = jnp.dot(emb, x, preferred_element_type=jnp.float32)
    e_sq = jnp.sum(emb * emb, axis=1, keepdims=True)
    x_sq = jnp.sum(x * x, axis=0, keepdims=True)
    dist = x_sq + e_sq - 2.0 * scores

    min_v = jnp.min(dist, axis=0, keepdims=True)
    iota_k = jax.lax.broadcasted_iota(jnp.int32, (K, N), 0)
    inds = jnp.min(jnp.where(dist == min_v, iota_k, K), axis=0,
                   keepdims=True)

    one_hot = (iota_k == inds).astype(jnp.float32)
    quantized = jax.lax.dot_general(
        emb, one_hot, (((0,), (0,)), ((), ())),
        preferred_element_type=jnp.float32)

    q_ref[0] = quantized
    inds_ref[...] = inds.reshape(1, 1, N)
    diff = quantized - x
    loss_ref[...] = jnp.sum(diff * diff).reshape(1, 1, 1)


def kernel(latents, emb):
    x3 = latents.reshape(B, D, N)
    q3, inds, loss_parts = pl.pallas_call(
        _vq_kernel,
        grid=(B,),
        in_specs=[
            pl.BlockSpec((1, D, N), lambda b: (b, 0, 0)),
            pl.BlockSpec((K, D), lambda b: (0, 0)),
        ],
        out_specs=[
            pl.BlockSpec((1, D, N), lambda b: (b, 0, 0)),
            pl.BlockSpec((1, 1, N), lambda b: (b, 0, 0)),
            pl.BlockSpec((1, 1, 1), lambda b: (b, 0, 0)),
        ],
        out_shape=[
            jax.ShapeDtypeStruct((B, D, N), jnp.float32),
            jax.ShapeDtypeStruct((B, 1, N), jnp.int32),
            jax.ShapeDtypeStruct((B, 1, 1), jnp.float32),
        ],
        compiler_params=pltpu.CompilerParams(
            dimension_semantics=("parallel",),
        ),
    )(x3, emb)
    loss = jnp.sum(loss_parts) / (B * N * D)
    q_out = q3.reshape(B, D, H, W_SP)
    encoding_inds = inds.reshape(B * N)
    return (q_out, loss, loss, encoding_inds)


# speedup vs baseline: 1.4506x; 1.0188x over previous
import jax
import jax.numpy as jnp
from jax.experimental import pallas as pl
from jax.experimental.pallas import tpu as pltpu

B, D, H, W_SP = 32, 64, 32, 32
K = 1024
N = H * W_SP
IMGS = 2
GRID = B // IMGS


def _vq_kernel(x_ref, emb_ref, q_ref, inds_ref, loss_ref):
    emb = emb_ref[...]
    e_sq = jnp.sum(emb * emb, axis=1, keepdims=True)
    loss_acc = jnp.float32(0.0)
    for i in range(IMGS):
        x = x_ref[i]
        scores = jnp.dot(emb, x, preferred_element_type=jnp.float32)
        x_sq = jnp.sum(x * x, axis=0, keepdims=True)
        dist = x_sq + e_sq - 2.0 * scores

        min_v = jnp.min(dist, axis=0, keepdims=True)
        iota_k = jax.lax.broadcasted_iota(jnp.int32, (K, N), 0)
        inds = jnp.min(jnp.where(dist == min_v, iota_k, K), axis=0,
                       keepdims=True)

        one_hot = (iota_k == inds).astype(jnp.float32)
        quantized = jax.lax.dot_general(
            emb, one_hot, (((0,), (0,)), ((), ())),
            preferred_element_type=jnp.float32)

        q_ref[i] = quantized
        inds_ref[0, i, :] = inds.reshape(N)
        diff = quantized - x
        loss_acc = loss_acc + jnp.sum(diff * diff)
    loss_ref[...] = loss_acc.reshape(1, 1, 1)


def kernel(latents, emb):
    x3 = latents.reshape(B, D, N)
    q3, inds, loss_parts = pl.pallas_call(
        _vq_kernel,
        grid=(GRID,),
        in_specs=[
            pl.BlockSpec((IMGS, D, N), lambda b: (b, 0, 0)),
            pl.BlockSpec((K, D), lambda b: (0, 0)),
        ],
        out_specs=[
            pl.BlockSpec((IMGS, D, N), lambda b: (b, 0, 0)),
            pl.BlockSpec((1, IMGS, N), lambda b: (b, 0, 0)),
            pl.BlockSpec((1, 1, 1), lambda b: (b, 0, 0)),
        ],
        out_shape=[
            jax.ShapeDtypeStruct((B, D, N), jnp.float32),
            jax.ShapeDtypeStruct((GRID, IMGS, N), jnp.int32),
            jax.ShapeDtypeStruct((GRID, 1, 1), jnp.float32),
        ],
        compiler_params=pltpu.CompilerParams(
            dimension_semantics=("parallel",),
        ),
    )(x3, emb)
    loss = jnp.sum(loss_parts) / (B * N * D)
    q_out = q3.reshape(B, D, H, W_SP)
    encoding_inds = inds.reshape(B * N)
    return (q_out, loss, loss, encoding_inds)


# speedup vs baseline: 1.5126x; 1.0428x over previous
import jax
import jax.numpy as jnp
from jax.experimental import pallas as pl
from jax.experimental.pallas import tpu as pltpu

B, D, H, W_SP = 32, 64, 32, 32
K = 1024
N = H * W_SP
IMGS = 4
GRID = B // IMGS


def _vq_kernel(x_ref, emb_ref, q_ref, inds_ref, loss_ref):
    emb = emb_ref[...]
    e_sq = jnp.sum(emb * emb, axis=1, keepdims=True)
    emb2 = emb + emb
    loss_acc = jnp.float32(0.0)
    for i in range(IMGS):
        x = x_ref[i]
        scores2 = jnp.dot(emb2, x, preferred_element_type=jnp.float32)
        x_sq = jnp.sum(x * x, axis=0, keepdims=True)
        dist = x_sq + e_sq - scores2

        min_v = jnp.min(dist, axis=0, keepdims=True)
        iota_k = jax.lax.broadcasted_iota(jnp.int32, (K, N), 0)
        inds = jnp.min(jnp.where(dist == min_v, iota_k, K), axis=0,
                       keepdims=True)

        one_hot = (iota_k == inds).astype(jnp.float32)
        quantized = jax.lax.dot_general(
            emb, one_hot, (((0,), (0,)), ((), ())),
            preferred_element_type=jnp.float32)

        q_ref[i] = quantized
        inds_ref[0, i, :] = inds.reshape(N)
        loss_acc = loss_acc + jnp.sum(min_v)
    loss_ref[...] = loss_acc.reshape(1, 1, 1)


def kernel(latents, emb):
    x3 = latents.reshape(B, D, N)
    q3, inds, loss_parts = pl.pallas_call(
        _vq_kernel,
        grid=(GRID,),
        in_specs=[
            pl.BlockSpec((IMGS, D, N), lambda b: (b, 0, 0)),
            pl.BlockSpec((K, D), lambda b: (0, 0)),
        ],
        out_specs=[
            pl.BlockSpec((IMGS, D, N), lambda b: (b, 0, 0)),
            pl.BlockSpec((1, IMGS, N), lambda b: (b, 0, 0)),
            pl.BlockSpec((1, 1, 1), lambda b: (b, 0, 0)),
        ],
        out_shape=[
            jax.ShapeDtypeStruct((B, D, N), jnp.float32),
            jax.ShapeDtypeStruct((GRID, IMGS, N), jnp.int32),
            jax.ShapeDtypeStruct((GRID, 1, 1), jnp.float32),
        ],
        compiler_params=pltpu.CompilerParams(
            dimension_semantics=("parallel",),
        ),
    )(x3, emb)
    loss = jnp.sum(loss_parts) / (B * N * D)
    q_out = q3.reshape(B, D, H, W_SP)
    encoding_inds = inds.reshape(B * N)
    return (q_out, loss, loss, encoding_inds)


# speedup vs baseline: 1.5233x; 1.0070x over previous
import jax
import jax.numpy as jnp
from jax.experimental import pallas as pl
from jax.experimental.pallas import tpu as pltpu

B, D, H, W_SP = 32, 64, 32, 32
K = 1024
N = H * W_SP
IMGS = 2
GRID = B // IMGS


def _vq_kernel(x_ref, emb_ref, q_ref, inds_ref, loss_ref):
    emb = emb_ref[...]
    e_sq = jnp.sum(emb * emb, axis=1, keepdims=True)
    emb2 = emb + emb
    loss_acc = jnp.float32(0.0)
    for i in range(IMGS):
        x = x_ref[i]
        scores2 = jnp.dot(emb2, x, preferred_element_type=jnp.float32)
        x_sq = jnp.sum(x * x, axis=0, keepdims=True)
        dist = x_sq + e_sq - scores2

        min_v = jnp.min(dist, axis=0, keepdims=True)
        iota_k = jax.lax.broadcasted_iota(jnp.int32, (K, N), 0)
        inds = jnp.min(jnp.where(dist == min_v, iota_k, K), axis=0,
                       keepdims=True)

        one_hot = (iota_k == inds).astype(jnp.float32)
        quantized = jax.lax.dot_general(
            emb, one_hot, (((0,), (0,)), ((), ())),
            preferred_element_type=jnp.float32)

        q_ref[i] = quantized
        inds_ref[0, i, :] = inds.reshape(N)
        loss_acc = loss_acc + jnp.sum(min_v)
    loss_ref[...] = loss_acc.reshape(1, 1, 1)


def kernel(latents, emb):
    x3 = latents.reshape(B, D, N)
    q3, inds, loss_parts = pl.pallas_call(
        _vq_kernel,
        grid=(GRID,),
        in_specs=[
            pl.BlockSpec((IMGS, D, N), lambda b: (b, 0, 0)),
            pl.BlockSpec((K, D), lambda b: (0, 0)),
        ],
        out_specs=[
            pl.BlockSpec((IMGS, D, N), lambda b: (b, 0, 0)),
            pl.BlockSpec((1, IMGS, N), lambda b: (b, 0, 0)),
            pl.BlockSpec((1, 1, 1), lambda b: (b, 0, 0)),
        ],
        out_shape=[
            jax.ShapeDtypeStruct((B, D, N), jnp.float32),
            jax.ShapeDtypeStruct((GRID, IMGS, N), jnp.int32),
            jax.ShapeDtypeStruct((GRID, 1, 1), jnp.float32),
        ],
        compiler_params=pltpu.CompilerParams(
            dimension_semantics=("parallel",),
        ),
    )(x3, emb)
    loss = jnp.sum(loss_parts) / (B * N * D)
    q_out = q3.reshape(B, D, H, W_SP)
    encoding_inds = inds.reshape(B * N)
    return (q_out, loss, loss, encoding_inds)
